# Initial kernel scaffold; baseline (speedup 1.0000x reference)
import jax
import jax.numpy as jnp
from jax.experimental import pallas as pl


def kernel(points, feats, W_sp, b_sp, theta_low, b_low, theta_high, b_high, W_g1, b_g1, ln_g1_gamma, ln_g1_beta, W_g2, b_g2, W_out, b_out, ln_out_gamma, ln_out_beta, gamma_res):
    raise NotImplementedError("write your pallas kernel here")


# speedup vs baseline: 11.2982x; 11.2982x over previous
import functools

import jax
import jax.numpy as jnp
from jax import lax
from jax.experimental import pallas as pl
from jax.experimental.pallas import tpu as pltpu
from jax.experimental.pallas import tpu_sc as plsc

_B, _N, _C, _KNN = 8, 2048, 128, 16
_NODES = _B * _N

_RB = 256
_NRB = _N // _RB


def _knn_body(pb_ref, pt_ref, idx_ref):
    b = pl.program_id(0)
    rb = pl.program_id(1)
    pb = pb_ref[0]
    pa = pt_ref[0]
    rb_p = pb.astype(jnp.bfloat16).astype(jnp.float32)
    ra_p = pa.astype(jnp.bfloat16).astype(jnp.float32)
    p0 = pb[:, 0:1]
    p1 = pb[:, 1:2]
    p2 = pb[:, 2:3]
    a0 = pa[0:1, :]
    a1 = pa[1:2, :]
    a2 = pa[2:3, :]
    sqb = p0 * p0 + p1 * p1 + p2 * p2
    sqa = a0 * a0 + a1 * a1 + a2 * a2
    dot = (rb_p[:, 0:1] * ra_p[0:1, :] + rb_p[:, 1:2] * ra_p[1:2, :]
           + rb_p[:, 2:3] * ra_p[2:3, :])
    d = sqb + sqa - 2.0 * dot
    colids = lax.broadcasted_iota(jnp.int32, (1, _N), 1)
    rowids = rb * _RB + lax.broadcasted_iota(jnp.int32, (_RB, 1), 0)
    d = jnp.where(colids == rowids, d + 1e10, d)
    base = b * _N
    for t in range(_KNN):
        m = jnp.min(d, axis=1, keepdims=True)
        sel = jnp.where(d == m, colids, jnp.int32(2 * _N))
        j = jnp.min(sel, axis=1, keepdims=True)
        idx_ref[0, :, t : t + 1] = j + base
        d = jnp.where(colids == j, jnp.float32(jnp.inf), d)


def _knn(points, points_t):
    return pl.pallas_call(
        _knn_body,
        grid=(_B, _NRB),
        in_specs=[
            pl.BlockSpec((1, _RB, 3), lambda b, r: (b, r, 0)),
            pl.BlockSpec((1, 3, _N), lambda b, r: (b, 0, 0)),
        ],
        out_specs=pl.BlockSpec((1, _RB, _KNN), lambda b, r: (b, r, 0)),
        out_shape=jax.ShapeDtypeStruct((_B, _N, _KNN), jnp.int32),
    )(points, points_t)


_NTILES = 32
_NPW = _NODES // _NTILES
_CH = 8
_NCH = _NPW // _CH
_IDXROWS = _NODES * _KNN // 128
_IRPW = _IDXROWS // _NTILES


@functools.lru_cache(maxsize=None)
def _make_sc_lap(alpha, bcoef, use_aux, ccoef):
    mesh = plsc.VectorSubcoreMesh(core_axis_name="c", subcore_axis_name="s",
                                  num_cores=2, num_subcores=16)

    def body(*refs):
        if use_aux:
            (x_hbm, aux_hbm, idx_hbm, out_hbm,
             idx_v, rows_v, xb_v, auxb_v, ob_v, sem) = refs
        else:
            (x_hbm, idx_hbm, out_hbm,
             idx_v, rows_v, xb_v, ob_v, sem) = refs
            aux_hbm = auxb_v = None
        wid = lax.axis_index("s") * 2 + lax.axis_index("c")
        pltpu.sync_copy(idx_hbm.at[pl.ds(wid * _IRPW, _IRPW)], idx_v)

        def chunk(g, carry):
            node_base = wid * _NPW + g * _CH
            pltpu.async_copy(x_hbm.at[idx_v.at[g]], rows_v, sem).wait()
            pltpu.sync_copy(x_hbm.at[pl.ds(node_base, _CH)], xb_v)
            if use_aux:
                pltpu.sync_copy(aux_hbm.at[pl.ds(node_base, _CH)], auxb_v)
            for n in range(_CH):
                for c8 in range(8):
                    sl = pl.ds(c8 * 16, 16)
                    s = rows_v[n * _KNN, sl]
                    for t in range(1, _KNN):
                        s = s + rows_v[n * _KNN + t, sl]
                    o = alpha * xb_v[n, sl] + bcoef * s
                    if use_aux:
                        o = o + ccoef * auxb_v[n, sl]
                    ob_v[n, sl] = o
            pltpu.sync_copy(ob_v, out_hbm.at[pl.ds(node_base, _CH)])
            return carry

        lax.fori_loop(0, _NCH, chunk, 0)

    scratch = [
        pltpu.VMEM((_IRPW, 128), jnp.int32),
        pltpu.VMEM((_CH * _KNN, _C), jnp.float32),
        pltpu.VMEM((_CH, _C), jnp.float32),
    ]
    if use_aux:
        scratch.append(pltpu.VMEM((_CH, _C), jnp.float32))
    scratch.append(pltpu.VMEM((_CH, _C), jnp.float32))
    scratch.append(pltpu.SemaphoreType.DMA)

    return pl.kernel(
        body,
        out_type=jax.ShapeDtypeStruct((_NODES, _C), jnp.float32),
        mesh=mesh,
        scratch_types=scratch,
    )


_RB2 = 512
_NRB2 = _NODES // _RB2


def _dense_body(t0_ref, t1_ref, t2_ref,
                wsp_ref, bsp_ref, thl0_ref, thl1_ref, thl2_ref, blo_ref,
                thh0_ref, thh1_ref, thh2_ref, bhi_ref,
                wg1a_ref, wg1b_ref, wg1c_ref, bg1_ref, g1g_ref, g1b_ref,
                wg2_ref, bg2_ref, wout_ref, bout_ref, og_ref, ob_ref,
                gres_ref, out_ref):
    f32 = jnp.float32
    x0 = t0_ref[...]
    x1 = t1_ref[...]
    x2 = t2_ref[...]
    dot = functools.partial(jnp.dot, preferred_element_type=f32)
    f_sp = dot(x0, wsp_ref[...]) + bsp_ref[...]
    f_lo = (dot(x0, thl0_ref[...]) + dot(x1, thl1_ref[...])
            + dot(x2, thl2_ref[...]) + blo_ref[...])
    f_hi = (dot(x0, thh0_ref[...]) + dot(x1, thh1_ref[...])
            + dot(x2, thh2_ref[...]) + bhi_ref[...])
    h = (dot(f_sp, wg1a_ref[...]) + dot(f_lo, wg1b_ref[...])
         + dot(f_hi, wg1c_ref[...]) + bg1_ref[...])
    mu = jnp.mean(h, axis=-1, keepdims=True)
    var = jnp.mean((h - mu) * (h - mu), axis=-1, keepdims=True)
    h = (h - mu) / jnp.sqrt(var + 1e-5) * g1g_ref[...] + g1b_ref[...]
    h = jnp.maximum(h, 0.0)
    g = dot(h, wg2_ref[...]) + bg2_ref[...]
    g0 = g[:, 0:1]
    g1 = g[:, 1:2]
    g2 = g[:, 2:3]
    m = jnp.maximum(jnp.maximum(g0, g1), g2)
    e0 = jnp.exp(g0 - m)
    e1 = jnp.exp(g1 - m)
    e2 = jnp.exp(g2 - m)
    inv = 1.0 / (e0 + e1 + e2)
    f_fused = (e0 * f_sp + e1 * f_lo + e2 * f_hi) * inv
    o = dot(f_fused, wout_ref[...]) + bout_ref[...]
    mu2 = jnp.mean(o, axis=-1, keepdims=True)
    var2 = jnp.mean((o - mu2) * (o - mu2), axis=-1, keepdims=True)
    o = (o - mu2) / jnp.sqrt(var2 + 1e-5) * og_ref[...] + ob_ref[...]
    out_ref[...] = x0 + gres_ref[0, 0] * o


def _dense(t0, t1, t2, *weights):
    row = pl.BlockSpec((_RB2, _C), lambda i: (i, 0))
    w128 = pl.BlockSpec((_C, _C), lambda i: (0, 0))
    vec = pl.BlockSpec((1, _C), lambda i: (0, 0))
    one = pl.BlockSpec((1, 1), lambda i: (0, 0))
    wspecs = [w128, vec, w128, w128, w128, vec, w128, w128, w128, vec,
              w128, w128, w128, vec, vec, vec, w128, vec, w128, vec,
              vec, vec, one]
    return pl.pallas_call(
        _dense_body,
        grid=(_NRB2,),
        in_specs=[row, row, row] + wspecs,
        out_specs=row,
        out_shape=jax.ShapeDtypeStruct((_NODES, _C), jnp.float32),
    )(t0, t1, t2, *weights)


def kernel(points, feats, W_sp, b_sp, theta_low, b_low, theta_high, b_high,
           W_g1, b_g1, ln_g1_gamma, ln_g1_beta, W_g2, b_g2, W_out, b_out,
           ln_out_gamma, ln_out_beta, gamma_res):
    points_t = jnp.transpose(points, (0, 2, 1))
    idx = _knn(points, points_t)
    idx_r = idx.reshape(_IDXROWS, 128)
    x = feats.reshape(_NODES, _C)
    t1 = _make_sc_lap(1.0, -1.0 / _KNN, False, 0.0)(x, idx_r)
    t2 = _make_sc_lap(2.0, -2.0 / _KNN, True, -1.0)(t1, x, idx_r)
    wg2p = jnp.zeros((_C, _C), jnp.float32).at[:, :3].set(W_g2)
    bg2p = jnp.zeros((1, _C), jnp.float32).at[0, :3].set(b_g2)
    r = lambda v: v.reshape(1, _C)
    out = _dense(
        x, t1, t2,
        W_sp, r(b_sp), theta_low[0], theta_low[1], theta_low[2], r(b_low),
        theta_high[0], theta_high[1], theta_high[2], r(b_high),
        W_g1[0:_C], W_g1[_C:2 * _C], W_g1[2 * _C:3 * _C], r(b_g1),
        r(ln_g1_gamma), r(ln_g1_beta), wg2p, bg2p, W_out, r(b_out),
        r(ln_out_gamma), r(ln_out_beta), gamma_res.reshape(1, 1),
    )
    return out.reshape(_B, _N, _C)


# speedup vs baseline: 12.3861x; 1.0963x over previous
import functools

import jax
import jax.numpy as jnp
from jax import lax
from jax.experimental import pallas as pl
from jax.experimental.pallas import tpu as pltpu
from jax.experimental.pallas import tpu_sc as plsc

_B, _N, _C, _KNN = 8, 2048, 128, 16
_NODES = _B * _N

_RB = 512
_NRB = _N // _RB


def _knn_body(pb_ref, pt_ref, idx_ref):
    b = pl.program_id(0)
    rb = pl.program_id(1)
    pb = pb_ref[0]
    pa = pt_ref[0]
    rb_p = pb.astype(jnp.bfloat16).astype(jnp.float32)
    ra_p = pa.astype(jnp.bfloat16).astype(jnp.float32)
    p0 = pb[:, 0:1]
    p1 = pb[:, 1:2]
    p2 = pb[:, 2:3]
    a0 = pa[0:1, :]
    a1 = pa[1:2, :]
    a2 = pa[2:3, :]
    sqb = p0 * p0 + p1 * p1 + p2 * p2
    sqa = a0 * a0 + a1 * a1 + a2 * a2
    dot = (rb_p[:, 0:1] * ra_p[0:1, :] + rb_p[:, 1:2] * ra_p[1:2, :]
           + rb_p[:, 2:3] * ra_p[2:3, :])
    d = sqb + sqa - 2.0 * dot
    colids = lax.broadcasted_iota(jnp.int32, (1, _N), 1)
    rowids = rb * _RB + lax.broadcasted_iota(jnp.int32, (_RB, 1), 0)
    d = jnp.where(colids == rowids, d + 1e10, d)
    base = b * _N
    for t in range(_KNN):
        m = jnp.min(d, axis=1, keepdims=True)
        eqm = d == m
        sel = jnp.where(eqm, colids, jnp.int32(2 * _N))
        j = jnp.min(sel, axis=1, keepdims=True)
        idx_ref[0, :, t : t + 1] = j + base
        d = jnp.where(eqm, jnp.float32(jnp.inf), d)


def _knn(points, points_t):
    return pl.pallas_call(
        _knn_body,
        grid=(_B, _NRB),
        in_specs=[
            pl.BlockSpec((1, _RB, 3), lambda b, r: (b, r, 0)),
            pl.BlockSpec((1, 3, _N), lambda b, r: (b, 0, 0)),
        ],
        out_specs=pl.BlockSpec((1, _RB, _KNN), lambda b, r: (b, r, 0)),
        out_shape=jax.ShapeDtypeStruct((_B, _N, _KNN), jnp.int32),
    )(points, points_t)


_NTILES = 32
_NPW = _NODES // _NTILES
_CH = 8
_NCH = _NPW // _CH
_IDXROWS = _NODES * _KNN // 128
_IRPW = _IDXROWS // _NTILES


@functools.lru_cache(maxsize=None)
def _make_sc_lap(alpha, bcoef, use_aux, ccoef):
    mesh = plsc.VectorSubcoreMesh(core_axis_name="c", subcore_axis_name="s",
                                  num_cores=2, num_subcores=16)

    def body(*refs):
        if use_aux:
            (x_hbm, aux_hbm, idx_hbm, out_hbm, idx_v,
             rows0_v, rows1_v, xb_v, auxb_v, ob_v, sem0, sem1) = refs
        else:
            (x_hbm, idx_hbm, out_hbm, idx_v,
             rows0_v, rows1_v, xb_v, ob_v, sem0, sem1) = refs
            aux_hbm = auxb_v = None
        rows = (rows0_v, rows1_v)
        sems = (sem0, sem1)
        wid = lax.axis_index("s") * 2 + lax.axis_index("c")
        pltpu.sync_copy(idx_hbm.at[pl.ds(wid * _IRPW, _IRPW)], idx_v)
        pltpu.async_copy(x_hbm.at[idx_v.at[0]], rows0_v, sem0)

        def do_chunk(g, buf):
            node_base = wid * _NPW + g * _CH
            pltpu.make_async_copy(
                x_hbm.at[idx_v.at[g]], rows[buf], sems[buf]).wait()
            pltpu.sync_copy(x_hbm.at[pl.ds(node_base, _CH)], xb_v)
            if use_aux:
                pltpu.sync_copy(aux_hbm.at[pl.ds(node_base, _CH)], auxb_v)
            for n in range(_CH):
                for c8 in range(8):
                    sl = pl.ds(c8 * 16, 16)
                    s = rows[buf][n * _KNN, sl]
                    for t in range(1, _KNN):
                        s = s + rows[buf][n * _KNN + t, sl]
                    o = alpha * xb_v[n, sl] + bcoef * s
                    if use_aux:
                        o = o + ccoef * auxb_v[n, sl]
                    ob_v[n, sl] = o
            pltpu.sync_copy(ob_v, out_hbm.at[pl.ds(node_base, _CH)])

        def pair(g2, carry):
            ga = 2 * g2
            pltpu.async_copy(x_hbm.at[idx_v.at[ga + 1]], rows1_v, sem1)
            do_chunk(ga, 0)

            @pl.when(ga + 2 < _NCH)
            def _():
                pltpu.async_copy(x_hbm.at[idx_v.at[ga + 2]], rows0_v, sem0)

            do_chunk(ga + 1, 1)
            return carry

        lax.fori_loop(0, _NCH // 2, pair, 0)

    scratch = [
        pltpu.VMEM((_IRPW, 128), jnp.int32),
        pltpu.VMEM((_CH * _KNN, _C), jnp.float32),
        pltpu.VMEM((_CH * _KNN, _C), jnp.float32),
        pltpu.VMEM((_CH, _C), jnp.float32),
    ]
    if use_aux:
        scratch.append(pltpu.VMEM((_CH, _C), jnp.float32))
    scratch.append(pltpu.VMEM((_CH, _C), jnp.float32))
    scratch.append(pltpu.SemaphoreType.DMA)
    scratch.append(pltpu.SemaphoreType.DMA)

    return pl.kernel(
        body,
        out_type=jax.ShapeDtypeStruct((_NODES, _C), jnp.float32),
        mesh=mesh,
        scratch_types=scratch,
    )


_RB2 = 512
_NRB2 = _NODES // _RB2


def _dense_body(t0_ref, t1_ref, t2_ref,
                wsp_ref, bsp_ref, thl0_ref, thl1_ref, thl2_ref, blo_ref,
                thh0_ref, thh1_ref, thh2_ref, bhi_ref,
                wg1a_ref, wg1b_ref, wg1c_ref, bg1_ref, g1g_ref, g1b_ref,
                wg2_ref, bg2_ref, wout_ref, bout_ref, og_ref, ob_ref,
                gres_ref, out_ref):
    f32 = jnp.float32
    x0 = t0_ref[...]
    x1 = t1_ref[...]
    x2 = t2_ref[...]
    dot = functools.partial(jnp.dot, preferred_element_type=f32)
    f_sp = dot(x0, wsp_ref[...]) + bsp_ref[...]
    f_lo = (dot(x0, thl0_ref[...]) + dot(x1, thl1_ref[...])
            + dot(x2, thl2_ref[...]) + blo_ref[...])
    f_hi = (dot(x0, thh0_ref[...]) + dot(x1, thh1_ref[...])
            + dot(x2, thh2_ref[...]) + bhi_ref[...])
    h = (dot(f_sp, wg1a_ref[...]) + dot(f_lo, wg1b_ref[...])
         + dot(f_hi, wg1c_ref[...]) + bg1_ref[...])
    mu = jnp.mean(h, axis=-1, keepdims=True)
    var = jnp.mean((h - mu) * (h - mu), axis=-1, keepdims=True)
    h = (h - mu) / jnp.sqrt(var + 1e-5) * g1g_ref[...] + g1b_ref[...]
    h = jnp.maximum(h, 0.0)
    g = dot(h, wg2_ref[...]) + bg2_ref[...]
    g0 = g[:, 0:1]
    g1 = g[:, 1:2]
    g2 = g[:, 2:3]
    m = jnp.maximum(jnp.maximum(g0, g1), g2)
    e0 = jnp.exp(g0 - m)
    e1 = jnp.exp(g1 - m)
    e2 = jnp.exp(g2 - m)
    inv = 1.0 / (e0 + e1 + e2)
    f_fused = (e0 * f_sp + e1 * f_lo + e2 * f_hi) * inv
    o = dot(f_fused, wout_ref[...]) + bout_ref[...]
    mu2 = jnp.mean(o, axis=-1, keepdims=True)
    var2 = jnp.mean((o - mu2) * (o - mu2), axis=-1, keepdims=True)
    o = (o - mu2) / jnp.sqrt(var2 + 1e-5) * og_ref[...] + ob_ref[...]
    out_ref[...] = x0 + gres_ref[0, 0] * o


def _dense(t0, t1, t2, *weights):
    row = pl.BlockSpec((_RB2, _C), lambda i: (i, 0))
    w128 = pl.BlockSpec((_C, _C), lambda i: (0, 0))
    vec = pl.BlockSpec((1, _C), lambda i: (0, 0))
    one = pl.BlockSpec((1, 1), lambda i: (0, 0))
    wspecs = [w128, vec, w128, w128, w128, vec, w128, w128, w128, vec,
              w128, w128, w128, vec, vec, vec, w128, vec, w128, vec,
              vec, vec, one]
    return pl.pallas_call(
        _dense_body,
        grid=(_NRB2,),
        in_specs=[row, row, row] + wspecs,
        out_specs=row,
        out_shape=jax.ShapeDtypeStruct((_NODES, _C), jnp.float32),
    )(t0, t1, t2, *weights)


def kernel(points, feats, W_sp, b_sp, theta_low, b_low, theta_high, b_high,
           W_g1, b_g1, ln_g1_gamma, ln_g1_beta, W_g2, b_g2, W_out, b_out,
           ln_out_gamma, ln_out_beta, gamma_res):
    points_t = jnp.transpose(points, (0, 2, 1))
    idx = _knn(points, points_t)
    idx_r = idx.reshape(_IDXROWS, 128)
    x = feats.reshape(_NODES, _C)
    t1 = _make_sc_lap(1.0, -1.0 / _KNN, False, 0.0)(x, idx_r)
    t2 = _make_sc_lap(2.0, -2.0 / _KNN, True, -1.0)(t1, x, idx_r)
    wg2p = jnp.zeros((_C, _C), jnp.float32).at[:, :3].set(W_g2)
    bg2p = jnp.zeros((1, _C), jnp.float32).at[0, :3].set(b_g2)
    r = lambda v: v.reshape(1, _C)
    out = _dense(
        x, t1, t2,
        W_sp, r(b_sp), theta_low[0], theta_low[1], theta_low[2], r(b_low),
        theta_high[0], theta_high[1], theta_high[2], r(b_high),
        W_g1[0:_C], W_g1[_C:2 * _C], W_g1[2 * _C:3 * _C], r(b_g1),
        r(ln_g1_gamma), r(ln_g1_beta), wg2p, bg2p, W_out, r(b_out),
        r(ln_out_gamma), r(ln_out_beta), gamma_res.reshape(1, 1),
    )
    return out.reshape(_B, _N, _C)


# speedup vs baseline: 12.8929x; 1.0409x over previous
import functools

import jax
import jax.numpy as jnp
from jax import lax
from jax.experimental import pallas as pl
from jax.experimental.pallas import tpu as pltpu
from jax.experimental.pallas import tpu_sc as plsc

_B, _N, _C, _KNN = 8, 2048, 128, 16
_NODES = _B * _N

_RB = 512
_NRB = _N // _RB


def _knn_body(pb_ref, pt_ref, idx_ref):
    b = pl.program_id(0)
    rb = pl.program_id(1)
    pb = pb_ref[0]
    pa = pt_ref[0]
    rb_p = pb.astype(jnp.bfloat16).astype(jnp.float32)
    ra_p = pa.astype(jnp.bfloat16).astype(jnp.float32)
    p0 = pb[:, 0:1]
    p1 = pb[:, 1:2]
    p2 = pb[:, 2:3]
    a0 = pa[0:1, :]
    a1 = pa[1:2, :]
    a2 = pa[2:3, :]
    sqb = p0 * p0 + p1 * p1 + p2 * p2
    sqa = a0 * a0 + a1 * a1 + a2 * a2
    dot = (rb_p[:, 0:1] * ra_p[0:1, :] + rb_p[:, 1:2] * ra_p[1:2, :]
           + rb_p[:, 2:3] * ra_p[2:3, :])
    d = sqb + sqa - 2.0 * dot
    colids = lax.broadcasted_iota(jnp.int32, (1, _N), 1)
    rowids = rb * _RB + lax.broadcasted_iota(jnp.int32, (_RB, 1), 0)
    d = jnp.where(colids == rowids, d + 1e10, d)
    base = b * _N
    for t in range(_KNN):
        j = jnp.argmin(d, axis=1).astype(jnp.int32)[:, None]
        idx_ref[0, :, t : t + 1] = j + base
        d = jnp.where(colids == j, jnp.float32(jnp.inf), d)


def _knn(points, points_t):
    return pl.pallas_call(
        _knn_body,
        grid=(_B, _NRB),
        in_specs=[
            pl.BlockSpec((1, _RB, 3), lambda b, r: (b, r, 0)),
            pl.BlockSpec((1, 3, _N), lambda b, r: (b, 0, 0)),
        ],
        out_specs=pl.BlockSpec((1, _RB, _KNN), lambda b, r: (b, r, 0)),
        out_shape=jax.ShapeDtypeStruct((_B, _N, _KNN), jnp.int32),
    )(points, points_t)


_NTILES = 32
_NPW = _NODES // _NTILES
_CH = 8
_NCH = _NPW // _CH
_IDXROWS = _NODES * _KNN // 128
_IRPW = _IDXROWS // _NTILES


@functools.lru_cache(maxsize=None)
def _make_sc_lap(alpha, bcoef, use_aux, ccoef):
    mesh = plsc.VectorSubcoreMesh(core_axis_name="c", subcore_axis_name="s",
                                  num_cores=2, num_subcores=16)

    def body(*refs):
        if use_aux:
            (x_hbm, aux_hbm, idx_hbm, out_hbm, idx_v,
             rows0_v, rows1_v, xb_v, auxb_v, ob_v, sem0, sem1) = refs
        else:
            (x_hbm, idx_hbm, out_hbm, idx_v,
             rows0_v, rows1_v, xb_v, ob_v, sem0, sem1) = refs
            aux_hbm = auxb_v = None
        rows = (rows0_v, rows1_v)
        sems = (sem0, sem1)
        wid = lax.axis_index("s") * 2 + lax.axis_index("c")
        pltpu.sync_copy(idx_hbm.at[pl.ds(wid * _IRPW, _IRPW)], idx_v)
        pltpu.async_copy(x_hbm.at[idx_v.at[0]], rows0_v, sem0)

        def do_chunk(g, buf):
            node_base = wid * _NPW + g * _CH
            pltpu.make_async_copy(
                x_hbm.at[idx_v.at[g]], rows[buf], sems[buf]).wait()
            pltpu.sync_copy(x_hbm.at[pl.ds(node_base, _CH)], xb_v)
            if use_aux:
                pltpu.sync_copy(aux_hbm.at[pl.ds(node_base, _CH)], auxb_v)
            for n in range(_CH):
                for c8 in range(8):
                    sl = pl.ds(c8 * 16, 16)
                    vals = [rows[buf][n * _KNN + t, sl] for t in range(_KNN)]
                    while len(vals) > 1:
                        vals = [vals[i] + vals[i + 1]
                                for i in range(0, len(vals), 2)]
                    s = vals[0]
                    o = alpha * xb_v[n, sl] + bcoef * s
                    if use_aux:
                        o = o + ccoef * auxb_v[n, sl]
                    ob_v[n, sl] = o
            pltpu.sync_copy(ob_v, out_hbm.at[pl.ds(node_base, _CH)])

        def pair(g2, carry):
            ga = 2 * g2
            pltpu.async_copy(x_hbm.at[idx_v.at[ga + 1]], rows1_v, sem1)
            do_chunk(ga, 0)

            @pl.when(ga + 2 < _NCH)
            def _():
                pltpu.async_copy(x_hbm.at[idx_v.at[ga + 2]], rows0_v, sem0)

            do_chunk(ga + 1, 1)
            return carry

        lax.fori_loop(0, _NCH // 2, pair, 0)

    scratch = [
        pltpu.VMEM((_IRPW, 128), jnp.int32),
        pltpu.VMEM((_CH * _KNN, _C), jnp.float32),
        pltpu.VMEM((_CH * _KNN, _C), jnp.float32),
        pltpu.VMEM((_CH, _C), jnp.float32),
    ]
    if use_aux:
        scratch.append(pltpu.VMEM((_CH, _C), jnp.float32))
    scratch.append(pltpu.VMEM((_CH, _C), jnp.float32))
    scratch.append(pltpu.SemaphoreType.DMA)
    scratch.append(pltpu.SemaphoreType.DMA)

    return pl.kernel(
        body,
        out_type=jax.ShapeDtypeStruct((_NODES, _C), jnp.float32),
        mesh=mesh,
        scratch_types=scratch,
    )


_RB2 = 512
_NRB2 = _NODES // _RB2


def _dense_body(t0_ref, t1_ref, t2_ref,
                wsp_ref, bsp_ref, thl0_ref, thl1_ref, thl2_ref, blo_ref,
                thh0_ref, thh1_ref, thh2_ref, bhi_ref,
                wg1a_ref, wg1b_ref, wg1c_ref, bg1_ref, g1g_ref, g1b_ref,
                wg2_ref, bg2_ref, wout_ref, bout_ref, og_ref, ob_ref,
                gres_ref, out_ref):
    f32 = jnp.float32
    x0 = t0_ref[...]
    x1 = t1_ref[...]
    x2 = t2_ref[...]
    dot = functools.partial(jnp.dot, preferred_element_type=f32)
    f_sp = dot(x0, wsp_ref[...]) + bsp_ref[...]
    f_lo = (dot(x0, thl0_ref[...]) + dot(x1, thl1_ref[...])
            + dot(x2, thl2_ref[...]) + blo_ref[...])
    f_hi = (dot(x0, thh0_ref[...]) + dot(x1, thh1_ref[...])
            + dot(x2, thh2_ref[...]) + bhi_ref[...])
    h = (dot(f_sp, wg1a_ref[...]) + dot(f_lo, wg1b_ref[...])
         + dot(f_hi, wg1c_ref[...]) + bg1_ref[...])
    mu = jnp.mean(h, axis=-1, keepdims=True)
    var = jnp.mean((h - mu) * (h - mu), axis=-1, keepdims=True)
    h = (h - mu) / jnp.sqrt(var + 1e-5) * g1g_ref[...] + g1b_ref[...]
    h = jnp.maximum(h, 0.0)
    g = dot(h, wg2_ref[...]) + bg2_ref[...]
    g0 = g[:, 0:1]
    g1 = g[:, 1:2]
    g2 = g[:, 2:3]
    m = jnp.maximum(jnp.maximum(g0, g1), g2)
    e0 = jnp.exp(g0 - m)
    e1 = jnp.exp(g1 - m)
    e2 = jnp.exp(g2 - m)
    inv = 1.0 / (e0 + e1 + e2)
    f_fused = (e0 * f_sp + e1 * f_lo + e2 * f_hi) * inv
    o = dot(f_fused, wout_ref[...]) + bout_ref[...]
    mu2 = jnp.mean(o, axis=-1, keepdims=True)
    var2 = jnp.mean((o - mu2) * (o - mu2), axis=-1, keepdims=True)
    o = (o - mu2) / jnp.sqrt(var2 + 1e-5) * og_ref[...] + ob_ref[...]
    out_ref[...] = x0 + gres_ref[0, 0] * o


def _dense(t0, t1, t2, *weights):
    row = pl.BlockSpec((_RB2, _C), lambda i: (i, 0))
    w128 = pl.BlockSpec((_C, _C), lambda i: (0, 0))
    vec = pl.BlockSpec((1, _C), lambda i: (0, 0))
    one = pl.BlockSpec((1, 1), lambda i: (0, 0))
    wspecs = [w128, vec, w128, w128, w128, vec, w128, w128, w128, vec,
              w128, w128, w128, vec, vec, vec, w128, vec, w128, vec,
              vec, vec, one]
    return pl.pallas_call(
        _dense_body,
        grid=(_NRB2,),
        in_specs=[row, row, row] + wspecs,
        out_specs=row,
        out_shape=jax.ShapeDtypeStruct((_NODES, _C), jnp.float32),
    )(t0, t1, t2, *weights)


def kernel(points, feats, W_sp, b_sp, theta_low, b_low, theta_high, b_high,
           W_g1, b_g1, ln_g1_gamma, ln_g1_beta, W_g2, b_g2, W_out, b_out,
           ln_out_gamma, ln_out_beta, gamma_res):
    points_t = jnp.transpose(points, (0, 2, 1))
    idx = _knn(points, points_t)
    idx_r = idx.reshape(_IDXROWS, 128)
    x = feats.reshape(_NODES, _C)
    t1 = _make_sc_lap(1.0, -1.0 / _KNN, False, 0.0)(x, idx_r)
    t2 = _make_sc_lap(2.0, -2.0 / _KNN, True, -1.0)(t1, x, idx_r)
    wg2p = jnp.zeros((_C, _C), jnp.float32).at[:, :3].set(W_g2)
    bg2p = jnp.zeros((1, _C), jnp.float32).at[0, :3].set(b_g2)
    r = lambda v: v.reshape(1, _C)
    out = _dense(
        x, t1, t2,
        W_sp, r(b_sp), theta_low[0], theta_low[1], theta_low[2], r(b_low),
        theta_high[0], theta_high[1], theta_high[2], r(b_high),
        W_g1[0:_C], W_g1[_C:2 * _C], W_g1[2 * _C:3 * _C], r(b_g1),
        r(ln_g1_gamma), r(ln_g1_beta), wg2p, bg2p, W_out, r(b_out),
        r(ln_out_gamma), r(ln_out_beta), gamma_res.reshape(1, 1),
    )
    return out.reshape(_B, _N, _C)


# speedup vs baseline: 15.1797x; 1.1774x over previous
import functools

import jax
import jax.numpy as jnp
from jax import lax
from jax.experimental import pallas as pl
from jax.experimental.pallas import tpu as pltpu
from jax.experimental.pallas import tpu_sc as plsc

_B, _N, _C, _KNN = 8, 2048, 128, 16
_NODES = _B * _N

_RB = 512
_NRB = _N // _RB


def _knn_body(pb_ref, pt_ref, idx_ref):
    b = pl.program_id(0)
    rb = pl.program_id(1)
    pb = pb_ref[0]
    pa = pt_ref[0]
    rb_p = pb.astype(jnp.bfloat16).astype(jnp.float32)
    ra_p = pa.astype(jnp.bfloat16).astype(jnp.float32)
    p0 = pb[:, 0:1]
    p1 = pb[:, 1:2]
    p2 = pb[:, 2:3]
    a0 = pa[0:1, :]
    a1 = pa[1:2, :]
    a2 = pa[2:3, :]
    sqb = p0 * p0 + p1 * p1 + p2 * p2
    sqa = a0 * a0 + a1 * a1 + a2 * a2
    dot = (rb_p[:, 0:1] * ra_p[0:1, :] + rb_p[:, 1:2] * ra_p[1:2, :]
           + rb_p[:, 2:3] * ra_p[2:3, :])
    d = sqb + sqa - 2.0 * dot
    colids = lax.broadcasted_iota(jnp.int32, (1, _N), 1)
    rowids = rb * _RB + lax.broadcasted_iota(jnp.int32, (_RB, 1), 0)
    d = jnp.where(colids == rowids, d + 1e10, d)
    base = b * _N
    for t in range(_KNN):
        j = jnp.argmin(d, axis=1).astype(jnp.int32)[:, None]
        idx_ref[0, :, t : t + 1] = j + base
        d = jnp.where(colids == j, jnp.float32(jnp.inf), d)


def _knn(points, points_t):
    return pl.pallas_call(
        _knn_body,
        grid=(_B, _NRB),
        in_specs=[
            pl.BlockSpec((1, _RB, 3), lambda b, r: (b, r, 0)),
            pl.BlockSpec((1, 3, _N), lambda b, r: (b, 0, 0)),
        ],
        out_specs=pl.BlockSpec((1, _RB, _KNN), lambda b, r: (b, r, 0)),
        out_shape=jax.ShapeDtypeStruct((_B, _N, _KNN), jnp.int32),
    )(points, points_t)


_NTILES = 32
_NPW = _NODES // _NTILES
_CH = 8
_NCH = _NPW // _CH
_IDXROWS = _NODES * _KNN // 128
_IRPW = _IDXROWS // _NTILES


@functools.lru_cache(maxsize=None)
def _make_sc_lap(alpha, bcoef, use_aux, ccoef):
    mesh = plsc.VectorSubcoreMesh(core_axis_name="c", subcore_axis_name="s",
                                  num_cores=2, num_subcores=16)

    def body(*refs):
        if use_aux:
            (x_hbm, aux_hbm, idx_hbm, out_hbm, idx_v,
             rows0_v, rows1_v, xb0_v, xb1_v, auxb0_v, auxb1_v,
             ob_v, sem0, sem1) = refs
            auxb = (auxb0_v, auxb1_v)
        else:
            (x_hbm, idx_hbm, out_hbm, idx_v,
             rows0_v, rows1_v, xb0_v, xb1_v, ob_v, sem0, sem1) = refs
            aux_hbm = None
            auxb = (None, None)
        rows = (rows0_v, rows1_v)
        xb = (xb0_v, xb1_v)
        sems = (sem0, sem1)
        wid = lax.axis_index("s") * 2 + lax.axis_index("c")
        pltpu.sync_copy(idx_hbm.at[pl.ds(wid * _IRPW, _IRPW)], idx_v)

        def start_in(g, buf):
            node_base = wid * _NPW + g * _CH
            pltpu.async_copy(x_hbm.at[idx_v.at[g]], rows[buf], sems[buf])
            pltpu.async_copy(
                x_hbm.at[pl.ds(node_base, _CH)], xb[buf], sems[buf])
            if use_aux:
                pltpu.async_copy(
                    aux_hbm.at[pl.ds(node_base, _CH)], auxb[buf], sems[buf])

        def do_chunk(g, buf):
            node_base = wid * _NPW + g * _CH
            pltpu.make_async_copy(
                x_hbm.at[idx_v.at[g]], rows[buf], sems[buf]).wait()
            pltpu.make_async_copy(
                x_hbm.at[pl.ds(node_base, _CH)], xb[buf], sems[buf]).wait()
            if use_aux:
                pltpu.make_async_copy(
                    aux_hbm.at[pl.ds(node_base, _CH)], auxb[buf],
                    sems[buf]).wait()
            for n in range(_CH):
                for c8 in range(8):
                    sl = pl.ds(c8 * 16, 16)
                    vals = [rows[buf][n * _KNN + t, sl] for t in range(_KNN)]
                    while len(vals) > 1:
                        vals = [vals[i] + vals[i + 1]
                                for i in range(0, len(vals), 2)]
                    s = vals[0]
                    o = alpha * xb[buf][n, sl] + bcoef * s
                    if use_aux:
                        o = o + ccoef * auxb[buf][n, sl]
                    ob_v[n, sl] = o
            pltpu.sync_copy(ob_v, out_hbm.at[pl.ds(node_base, _CH)])

        start_in(0, 0)

        def pair(g2, carry):
            ga = 2 * g2
            start_in(ga + 1, 1)
            do_chunk(ga, 0)

            @pl.when(ga + 2 < _NCH)
            def _():
                start_in(ga + 2, 0)

            do_chunk(ga + 1, 1)
            return carry

        lax.fori_loop(0, _NCH // 2, pair, 0)

    scratch = [
        pltpu.VMEM((_IRPW, 128), jnp.int32),
        pltpu.VMEM((_CH * _KNN, _C), jnp.float32),
        pltpu.VMEM((_CH * _KNN, _C), jnp.float32),
        pltpu.VMEM((_CH, _C), jnp.float32),
        pltpu.VMEM((_CH, _C), jnp.float32),
    ]
    if use_aux:
        scratch.append(pltpu.VMEM((_CH, _C), jnp.float32))
        scratch.append(pltpu.VMEM((_CH, _C), jnp.float32))
    scratch.append(pltpu.VMEM((_CH, _C), jnp.float32))
    scratch.append(pltpu.SemaphoreType.DMA)
    scratch.append(pltpu.SemaphoreType.DMA)

    return pl.kernel(
        body,
        out_type=jax.ShapeDtypeStruct((_NODES, _C), jnp.float32),
        mesh=mesh,
        scratch_types=scratch,
    )


_RB2 = 512
_NRB2 = _NODES // _RB2


def _dense_body(t0_ref, t1_ref, t2_ref,
                wsp_ref, bsp_ref, thl0_ref, thl1_ref, thl2_ref, blo_ref,
                thh0_ref, thh1_ref, thh2_ref, bhi_ref,
                wg1a_ref, wg1b_ref, wg1c_ref, bg1_ref, g1g_ref, g1b_ref,
                wg2_ref, bg2_ref, wout_ref, bout_ref, og_ref, ob_ref,
                gres_ref, out_ref):
    f32 = jnp.float32
    x0 = t0_ref[...]
    x1 = t1_ref[...]
    x2 = t2_ref[...]
    dot = functools.partial(jnp.dot, preferred_element_type=f32)
    f_sp = dot(x0, wsp_ref[...]) + bsp_ref[...]
    f_lo = (dot(x0, thl0_ref[...]) + dot(x1, thl1_ref[...])
            + dot(x2, thl2_ref[...]) + blo_ref[...])
    f_hi = (dot(x0, thh0_ref[...]) + dot(x1, thh1_ref[...])
            + dot(x2, thh2_ref[...]) + bhi_ref[...])
    h = (dot(f_sp, wg1a_ref[...]) + dot(f_lo, wg1b_ref[...])
         + dot(f_hi, wg1c_ref[...]) + bg1_ref[...])
    mu = jnp.mean(h, axis=-1, keepdims=True)
    var = jnp.mean((h - mu) * (h - mu), axis=-1, keepdims=True)
    h = (h - mu) / jnp.sqrt(var + 1e-5) * g1g_ref[...] + g1b_ref[...]
    h = jnp.maximum(h, 0.0)
    g = dot(h, wg2_ref[...]) + bg2_ref[...]
    g0 = g[:, 0:1]
    g1 = g[:, 1:2]
    g2 = g[:, 2:3]
    m = jnp.maximum(jnp.maximum(g0, g1), g2)
    e0 = jnp.exp(g0 - m)
    e1 = jnp.exp(g1 - m)
    e2 = jnp.exp(g2 - m)
    inv = 1.0 / (e0 + e1 + e2)
    f_fused = (e0 * f_sp + e1 * f_lo + e2 * f_hi) * inv
    o = dot(f_fused, wout_ref[...]) + bout_ref[...]
    mu2 = jnp.mean(o, axis=-1, keepdims=True)
    var2 = jnp.mean((o - mu2) * (o - mu2), axis=-1, keepdims=True)
    o = (o - mu2) / jnp.sqrt(var2 + 1e-5) * og_ref[...] + ob_ref[...]
    out_ref[...] = x0 + gres_ref[0, 0] * o


def _dense(t0, t1, t2, *weights):
    row = pl.BlockSpec((_RB2, _C), lambda i: (i, 0))
    w128 = pl.BlockSpec((_C, _C), lambda i: (0, 0))
    vec = pl.BlockSpec((1, _C), lambda i: (0, 0))
    one = pl.BlockSpec((1, 1), lambda i: (0, 0))
    wspecs = [w128, vec, w128, w128, w128, vec, w128, w128, w128, vec,
              w128, w128, w128, vec, vec, vec, w128, vec, w128, vec,
              vec, vec, one]
    return pl.pallas_call(
        _dense_body,
        grid=(_NRB2,),
        in_specs=[row, row, row] + wspecs,
        out_specs=row,
        out_shape=jax.ShapeDtypeStruct((_NODES, _C), jnp.float32),
    )(t0, t1, t2, *weights)


def kernel(points, feats, W_sp, b_sp, theta_low, b_low, theta_high, b_high,
           W_g1, b_g1, ln_g1_gamma, ln_g1_beta, W_g2, b_g2, W_out, b_out,
           ln_out_gamma, ln_out_beta, gamma_res):
    points_t = jnp.transpose(points, (0, 2, 1))
    idx = _knn(points, points_t)
    idx_r = idx.reshape(_IDXROWS, 128)
    x = feats.reshape(_NODES, _C)
    t1 = _make_sc_lap(1.0, -1.0 / _KNN, False, 0.0)(x, idx_r)
    t2 = _make_sc_lap(2.0, -2.0 / _KNN, True, -1.0)(t1, x, idx_r)
    wg2p = jnp.zeros((_C, _C), jnp.float32).at[:, :3].set(W_g2)
    bg2p = jnp.zeros((1, _C), jnp.float32).at[0, :3].set(b_g2)
    r = lambda v: v.reshape(1, _C)
    out = _dense(
        x, t1, t2,
        W_sp, r(b_sp), theta_low[0], theta_low[1], theta_low[2], r(b_low),
        theta_high[0], theta_high[1], theta_high[2], r(b_high),
        W_g1[0:_C], W_g1[_C:2 * _C], W_g1[2 * _C:3 * _C], r(b_g1),
        r(ln_g1_gamma), r(ln_g1_beta), wg2p, bg2p, W_out, r(b_out),
        r(ln_out_gamma), r(ln_out_beta), gamma_res.reshape(1, 1),
    )
    return out.reshape(_B, _N, _C)
